# initial kernel scaffold (unmeasured)
import jax
import jax.numpy as jnp
from jax import lax
from jax.experimental import pallas as pl
from jax.experimental.pallas import tpu as pltpu

N_DEV = 32


def kernel(x, w_mat):
    m_per, k = x.shape
    _, n_per = w_mat.shape

    def body(x_ref, w_ref, out_ref, xg_ref, send_sems, recv_sems,
             abuf_ref, amax_send_sems, amax_recv_sems):
        my = lax.axis_index("i")
        right = (my + 1) % N_DEV

        xg_ref[my] = x_ref[...]

        def gemm(origin):
            blk = jnp.dot(
                xg_ref[origin], w_ref[...],
                preferred_element_type=jnp.float32,
                precision=lax.Precision.HIGHEST,
            )
            out_ref[pl.ds(origin * m_per, m_per), :] = blk
            return jnp.max(jnp.abs(blk))

        amax = jnp.zeros((), jnp.float32)
        for h in range(N_DEV - 1):
            o = (my + N_DEV - h) % N_DEV
            rdma = pltpu.make_async_remote_copy(
                src_ref=xg_ref.at[o],
                dst_ref=xg_ref.at[o],
                send_sem=send_sems.at[h],
                recv_sem=recv_sems.at[h],
                device_id=(right,),
                device_id_type=pl.DeviceIdType.MESH,
            )
            rdma.start()
            amax = jnp.maximum(amax, gemm(o))
            rdma.wait()
        amax = jnp.maximum(amax, gemm((my + 1) % N_DEV))

        for r in range(5):
            partner = my ^ (1 << r)
            abuf_ref[0] = jnp.broadcast_to(amax, (8, 128))
            rdma = pltpu.make_async_remote_copy(
                src_ref=abuf_ref.at[0],
                dst_ref=abuf_ref.at[r + 1],
                send_sem=amax_send_sems.at[r],
                recv_sem=amax_recv_sems.at[r],
                device_id=(partner,),
                device_id_type=pl.DeviceIdType.MESH,
            )
            rdma.start()
            rdma.wait()
            amax = jnp.maximum(amax, abuf_ref[r + 1, 0, 0])

        scale = amax / 127.0
        y = out_ref[...]
        out_ref[...] = jnp.clip(jnp.round(y / scale), -127.0, 127.0) * scale

    return pl.pallas_call(
        body,
        out_shape=jax.ShapeDtypeStruct((N_DEV * m_per, n_per), jnp.float32),
        in_specs=[
            pl.BlockSpec(memory_space=pltpu.VMEM),
            pl.BlockSpec(memory_space=pltpu.VMEM),
        ],
        out_specs=pl.BlockSpec(memory_space=pltpu.VMEM),
        scratch_shapes=[
            pltpu.VMEM((N_DEV, m_per, k), jnp.float32),
            pltpu.SemaphoreType.DMA((N_DEV - 1,)),
            pltpu.SemaphoreType.DMA((N_DEV - 1,)),
            pltpu.VMEM((6, 8, 128), jnp.float32),
            pltpu.SemaphoreType.DMA((5,)),
            pltpu.SemaphoreType.DMA((5,)),
        ],
        compiler_params=pltpu.CompilerParams(
            vmem_limit_bytes=128 * 1024 * 1024,
        ),
    )(x, w_mat)


# baseline (device time: 791123 ns/iter reference)
import os

import jax
import jax.numpy as jnp
from jax import lax
from jax.experimental import pallas as pl
from jax.experimental.pallas import tpu as pltpu

_VARIANT = os.environ.get("T_VARIANT", "full")

N_DEV = 32
N_SLOTS = 16


def kernel(x, w_mat):
    m_per, k = x.shape
    _, n_per = w_mat.shape

    def body(x_ref, w_ref, out_ref, xg_ref, send_sems, recv_sems,
             credit_sem, abuf_ref, amax_send_sems, amax_recv_sems):
        my = lax.axis_index("i")
        right = (my + 1) % N_DEV
        left = (my + N_DEV - 1) % N_DEV

        xg_ref[my % N_SLOTS] = x_ref[...]

        def gemm(origin, slot):
            blk = jnp.dot(
                xg_ref[slot], w_ref[...],
                preferred_element_type=jnp.float32,
                precision=lax.Precision.HIGHEST,
            )
            out_ref[pl.ds(origin * m_per, m_per), :] = blk
            return jnp.max(jnp.abs(blk))

        def hop(h, amax):
            o = (my + N_DEV - h) % N_DEV
            slot = (my + N_DEV - h) % N_SLOTS

            @pl.when(h >= N_SLOTS - 1)
            def _():
                pl.semaphore_wait(credit_sem, 1)

            rdma = pltpu.make_async_remote_copy(
                src_ref=xg_ref.at[slot],
                dst_ref=xg_ref.at[slot],
                send_sem=send_sems.at[h],
                recv_sem=recv_sems.at[h],
                device_id=(right,),
                device_id_type=pl.DeviceIdType.MESH,
            )
            rdma.start()
            amax = jnp.maximum(amax, gemm(o, slot))
            rdma.wait()

            @pl.when(h <= N_DEV - 1 - N_SLOTS)
            def _():
                pl.semaphore_signal(
                    credit_sem, inc=1,
                    device_id=(left,),
                    device_id_type=pl.DeviceIdType.MESH,
                )

            return amax

        amax = lax.fori_loop(0, N_DEV - 1, hop, jnp.zeros((), jnp.float32))
        o_last = (my + 1) % N_DEV
        amax = jnp.maximum(amax, gemm(o_last, o_last % N_SLOTS))

        for r in range(5):
            partner = my ^ (1 << r)
            abuf_ref[0] = jnp.broadcast_to(amax, (8, 128))
            rdma = pltpu.make_async_remote_copy(
                src_ref=abuf_ref.at[0],
                dst_ref=abuf_ref.at[r + 1],
                send_sem=amax_send_sems.at[r],
                recv_sem=amax_recv_sems.at[r],
                device_id=(partner,),
                device_id_type=pl.DeviceIdType.MESH,
            )
            rdma.start()
            rdma.wait()
            amax = jnp.maximum(amax, abuf_ref[r + 1, 0, 0])

        scale = amax / 127.0
        y = out_ref[...]
        out_ref[...] = jnp.clip(jnp.round(y / scale), -127.0, 127.0) * scale

    return pl.pallas_call(
        body,
        out_shape=jax.ShapeDtypeStruct((N_DEV * m_per, n_per), jnp.float32),
        in_specs=[
            pl.BlockSpec(memory_space=pltpu.VMEM),
            pl.BlockSpec(memory_space=pltpu.VMEM),
        ],
        out_specs=pl.BlockSpec(memory_space=pltpu.VMEM),
        scratch_shapes=[
            pltpu.VMEM((N_SLOTS, m_per, k), jnp.float32),
            pltpu.SemaphoreType.DMA((N_DEV - 1,)),
            pltpu.SemaphoreType.DMA((N_DEV - 1,)),
            pltpu.SemaphoreType.REGULAR,
            pltpu.VMEM((6, 8, 128), jnp.float32),
            pltpu.SemaphoreType.DMA((5,)),
            pltpu.SemaphoreType.DMA((5,)),
        ],
        compiler_params=pltpu.CompilerParams(
            vmem_limit_bytes=128 * 1024 * 1024,
        ),
    )(x, w_mat)


# device time: 782475 ns/iter; 1.0111x vs baseline; 1.0111x over previous
import jax
import jax.numpy as jnp
from jax import lax
from jax.experimental import pallas as pl
from jax.experimental.pallas import tpu as pltpu

N_DEV = 32
N_SLOTS = 8
R_HOPS = 16
L_HOPS = 15


def kernel(x, w_mat):
    m_per, k = x.shape
    _, n_per = w_mat.shape

    def body(x_ref, w_ref, out_ref, rbuf, lbuf,
             rsend, rrecv, lsend, lrecv, credit_r, credit_l,
             abuf_ref, asend, arecv):
        my = lax.axis_index("i")
        right = (my + 1) % N_DEV
        left = (my + N_DEV - 1) % N_DEV

        def gemm(origin, src_ref):
            blk = jnp.dot(
                src_ref[...], w_ref[...],
                preferred_element_type=jnp.float32,
                precision=lax.Precision.HIGHEST,
            )
            out_ref[pl.ds(origin * m_per, m_per), :] = blk
            return jnp.max(jnp.abs(blk))

        def right_rdma(h, src_ref):
            return pltpu.make_async_remote_copy(
                src_ref=src_ref,
                dst_ref=rbuf.at[h % N_SLOTS],
                send_sem=rsend.at[h],
                recv_sem=rrecv.at[h],
                device_id=(right,),
                device_id_type=pl.DeviceIdType.MESH,
            )

        def left_rdma(h, src_ref):
            return pltpu.make_async_remote_copy(
                src_ref=src_ref,
                dst_ref=lbuf.at[h % N_SLOTS],
                send_sem=lsend.at[h],
                recv_sem=lrecv.at[h],
                device_id=(left,),
                device_id_type=pl.DeviceIdType.MESH,
            )

        r0 = right_rdma(0, x_ref)
        l0 = left_rdma(0, x_ref)
        r0.start()
        l0.start()
        amax = gemm(my, x_ref)
        r0.wait()
        l0.wait()

        def hop(h, amax):
            @pl.when(h >= N_SLOTS)
            def _():
                pl.semaphore_wait(credit_r, 1)
                pl.semaphore_wait(credit_l, 1)

            rd = right_rdma(h, rbuf.at[(h - 1) % N_SLOTS])
            ld = left_rdma(h, lbuf.at[(h - 1) % N_SLOTS])
            rd.start()
            ld.start()
            amax = jnp.maximum(
                amax, gemm((my + N_DEV - h) % N_DEV, rbuf.at[(h - 1) % N_SLOTS]))
            amax = jnp.maximum(
                amax, gemm((my + h) % N_DEV, lbuf.at[(h - 1) % N_SLOTS]))
            rd.wait()
            ld.wait()

            @pl.when(h <= R_HOPS - N_SLOTS)
            def _():
                pl.semaphore_signal(
                    credit_r, inc=1,
                    device_id=(left,), device_id_type=pl.DeviceIdType.MESH)

            @pl.when(h <= L_HOPS - N_SLOTS)
            def _():
                pl.semaphore_signal(
                    credit_l, inc=1,
                    device_id=(right,), device_id_type=pl.DeviceIdType.MESH)

            return amax

        amax = lax.fori_loop(1, L_HOPS, hop, amax)

        pl.semaphore_wait(credit_r, 1)
        r15 = right_rdma(R_HOPS - 1, rbuf.at[(R_HOPS - 2) % N_SLOTS])
        r15.start()
        amax = jnp.maximum(
            amax, gemm((my + N_DEV - (R_HOPS - 1)) % N_DEV,
                       rbuf.at[(R_HOPS - 2) % N_SLOTS]))
        amax = jnp.maximum(
            amax, gemm((my + L_HOPS) % N_DEV,
                       lbuf.at[(L_HOPS - 1) % N_SLOTS]))
        r15.wait()
        amax = jnp.maximum(
            amax, gemm((my + N_DEV - R_HOPS) % N_DEV,
                       rbuf.at[(R_HOPS - 1) % N_SLOTS]))

        for r in range(5):
            partner = my ^ (1 << r)
            abuf_ref[0] = jnp.broadcast_to(amax, (8, 128))
            rdma = pltpu.make_async_remote_copy(
                src_ref=abuf_ref.at[0],
                dst_ref=abuf_ref.at[r + 1],
                send_sem=asend.at[r],
                recv_sem=arecv.at[r],
                device_id=(partner,),
                device_id_type=pl.DeviceIdType.MESH,
            )
            rdma.start()
            rdma.wait()
            amax = jnp.maximum(amax, abuf_ref[r + 1, 0, 0])

        scale = amax / 127.0
        y = out_ref[...]
        out_ref[...] = jnp.clip(jnp.round(y / scale), -127.0, 127.0) * scale

    return pl.pallas_call(
        body,
        out_shape=jax.ShapeDtypeStruct((N_DEV * m_per, n_per), jnp.float32),
        in_specs=[
            pl.BlockSpec(memory_space=pltpu.VMEM),
            pl.BlockSpec(memory_space=pltpu.VMEM),
        ],
        out_specs=pl.BlockSpec(memory_space=pltpu.VMEM),
        scratch_shapes=[
            pltpu.VMEM((N_SLOTS, m_per, k), jnp.float32),
            pltpu.VMEM((N_SLOTS, m_per, k), jnp.float32),
            pltpu.SemaphoreType.DMA((R_HOPS,)),
            pltpu.SemaphoreType.DMA((R_HOPS,)),
            pltpu.SemaphoreType.DMA((L_HOPS,)),
            pltpu.SemaphoreType.DMA((L_HOPS,)),
            pltpu.SemaphoreType.REGULAR,
            pltpu.SemaphoreType.REGULAR,
            pltpu.VMEM((6, 8, 128), jnp.float32),
            pltpu.SemaphoreType.DMA((5,)),
            pltpu.SemaphoreType.DMA((5,)),
        ],
        compiler_params=pltpu.CompilerParams(
            vmem_limit_bytes=128 * 1024 * 1024,
        ),
    )(x, w_mat)
